# baseline (speedup 1.0000x reference)
import functools

import jax
import jax.numpy as jnp
from jax import lax
from jax.experimental import pallas as pl
from jax.experimental.pallas import tpu as pltpu
from jax.experimental.pallas import tpu_sc as plsc

B_ROWS = 4096
SEQ = 200
EMBED = 64
TOTAL = B_ROWS * SEQ

NC = 2
NS = 16
NW = NC * NS
PER_W = TOTAL // NW
CHUNK = 128
N_CHUNK = PER_W // CHUNK


@functools.partial(
    pl.kernel,
    mesh=plsc.VectorSubcoreMesh(core_axis_name="c", subcore_axis_name="s"),
    compiler_params=pltpu.CompilerParams(use_tc_tiling_on_sc=False),
    out_type=jax.ShapeDtypeStruct((TOTAL, EMBED), jnp.float32),
    scratch_types=[
        pltpu.VMEM((CHUNK,), jnp.int32),
        pltpu.VMEM((CHUNK, EMBED), jnp.float32),
        pltpu.SemaphoreType.DMA,
    ],
)
def _emb_lookup(x_hbm, table_hbm, out_hbm, idx_v, rows_v, sem):
    wid = lax.axis_index("s") * NC + lax.axis_index("c")
    base = wid * PER_W

    def body(i, carry):
        off = base + i * CHUNK
        pltpu.sync_copy(x_hbm.at[pl.ds(off, CHUNK)], idx_v)
        pltpu.async_copy(table_hbm.at[idx_v], rows_v, sem).wait()
        pltpu.sync_copy(rows_v, out_hbm.at[pl.ds(off, CHUNK)])
        return carry

    lax.fori_loop(0, N_CHUNK, body, 0)


def kernel(x, table):
    flat = x.reshape(TOTAL).astype(jnp.int32)
    out = _emb_lookup(flat, table)
    return out.reshape(B_ROWS, SEQ, EMBED)


# speedup vs baseline: 1.1932x; 1.1932x over previous
import functools

import jax
import jax.numpy as jnp
from jax import lax
from jax.experimental import pallas as pl
from jax.experimental.pallas import tpu as pltpu
from jax.experimental.pallas import tpu_sc as plsc

B_ROWS = 4096
SEQ = 200
EMBED = 64
TOTAL = B_ROWS * SEQ

NC = 2
NS = 16
NW = NC * NS
PER_W = TOTAL // NW
CHUNK = 128
N_CHUNK = PER_W // CHUNK
K = 8
N_GROUP = N_CHUNK // K


@functools.partial(
    pl.kernel,
    mesh=plsc.VectorSubcoreMesh(core_axis_name="c", subcore_axis_name="s"),
    compiler_params=pltpu.CompilerParams(use_tc_tiling_on_sc=False),
    out_type=jax.ShapeDtypeStruct((TOTAL, EMBED), jnp.float32),
    scratch_types=(
        [pltpu.VMEM((N_CHUNK, CHUNK), jnp.int32)]
        + [pltpu.VMEM((CHUNK, EMBED), jnp.float32) for _ in range(K)]
        + [pltpu.SemaphoreType.DMA for _ in range(2 * K)]
    ),
)
def _emb_lookup(x_hbm, table_hbm, out_hbm, idx_all, *rest):
    rows = rest[:K]
    gsem = rest[K : 2 * K]
    ssem = rest[2 * K :]
    wid = lax.axis_index("s") * NC + lax.axis_index("c")
    base = wid * PER_W

    pltpu.sync_copy(x_hbm.at[wid], idx_all)

    def group(i, carry):
        for b in range(K):
            c = i * K + b

            @pl.when(i > 0)
            def _wait_prev_store(b=b):
                pltpu.make_async_copy(
                    rows[b], out_hbm.at[pl.ds(0, CHUNK)], ssem[b]
                ).wait()

            pltpu.async_copy(table_hbm.at[idx_all.at[c]], rows[b], gsem[b])
        for b in range(K):
            c = i * K + b
            pltpu.make_async_copy(table_hbm.at[idx_all.at[c]], rows[b], gsem[b]).wait()
            pltpu.async_copy(rows[b], out_hbm.at[pl.ds(base + c * CHUNK, CHUNK)], ssem[b])
        return carry

    lax.fori_loop(0, N_GROUP, group, 0)
    for b in range(K):
        pltpu.make_async_copy(rows[b], out_hbm.at[pl.ds(0, CHUNK)], ssem[b]).wait()


def kernel(x, table):
    idx = x.reshape(NW, N_CHUNK, CHUNK).astype(jnp.int32)
    out = _emb_lookup(idx, table)
    return out.reshape(B_ROWS, SEQ, EMBED)
